# baseline (speedup 1.0000x reference)
import functools

import numpy as np

import jax
import jax.numpy as jnp
from jax import lax
from jax.experimental import pallas as pl
from jax.experimental.pallas import tpu as pltpu
from jax.experimental.pallas import tpu_sc as plsc

VOCAB = 100000
D = 768
B = 4
S = 2048
N = B * S
NC = 2
NS = 16
NW = NC * NS
TPW = N // NW
CH = 32
NCH = TPW // CH
LANES = 16
VEC = D // LANES


@functools.lru_cache(maxsize=1)
def _positional_encoding(max_seq, d_model):
    pos = np.arange(max_seq, dtype=np.float32)[:, None]
    i = np.arange(0, d_model, 2, dtype=np.float32)[None, :]
    denom = np.power(np.float32(10000.0), i / np.float32(d_model))
    arg = (pos / denom).astype(np.float32)
    pe = np.stack([np.sin(arg), np.cos(arg)], axis=2).astype(np.float32)
    return jnp.asarray(pe.reshape(max_seq * d_model))


def _body(table, tokens, pe, out, idx_v,
          rows0, rows1, pe0, pe1,
          sg0, sg1, sp0, sp1):
    rows = (rows0, rows1)
    pes = (pe0, pe1)
    sgs = (sg0, sg1)
    sps = (sp0, sp1)
    wid = lax.axis_index("s") * NC + lax.axis_index("c")
    base = wid * TPW
    pltpu.sync_copy(tokens.at[pl.ds(base, TPW)], idx_v)
    s0 = lax.rem(base, S)

    def start(c):
        i = c % 2
        cb = c * CH
        pltpu.async_copy(table.at[idx_v.at[pl.ds(cb, CH)]], rows[i], sgs[i])
        pltpu.async_copy(pe.at[pl.ds((s0 + cb) * D, CH * D)], pes[i], sps[i])

    start(0)
    start(1)
    for c in range(NCH):
        i = c % 2
        cb = c * CH
        pltpu.make_async_copy(table.at[idx_v.at[pl.ds(cb, CH)]],
                              rows[i], sgs[i]).wait()
        pltpu.make_async_copy(pe.at[pl.ds((s0 + cb) * D, CH * D)], pes[i],
                              sps[i]).wait()

        def add_row(t, carry):
            for j in range(VEC):
                rows[i][t, pl.ds(j * LANES, LANES)] = (
                    rows[i][t, pl.ds(j * LANES, LANES)]
                    + pes[i][pl.ds(t * D + j * LANES, LANES)])
            return carry

        lax.fori_loop(0, CH, add_row, 0)
        pltpu.sync_copy(rows[i], out.at[pl.ds(base + cb, CH)])
        if c + 2 < NCH:
            start(c + 2)


@jax.jit
def kernel(tokens, table):
    pe = _positional_encoding(S, D)
    tok = tokens.reshape(N).astype(jnp.int32)
    mesh = plsc.VectorSubcoreMesh(core_axis_name="c", subcore_axis_name="s")
    f = pl.kernel(
        _body,
        out_type=jax.ShapeDtypeStruct((N, D), jnp.float32),
        mesh=mesh,
        scratch_types=[
            pltpu.VMEM((TPW,), jnp.int32),
            pltpu.VMEM((CH, D), jnp.float32),
            pltpu.VMEM((CH, D), jnp.float32),
            pltpu.VMEM((CH * D,), jnp.float32),
            pltpu.VMEM((CH * D,), jnp.float32),
            pltpu.SemaphoreType.DMA,
            pltpu.SemaphoreType.DMA,
            pltpu.SemaphoreType.DMA,
            pltpu.SemaphoreType.DMA,
        ],
    )
    out = f(table, tok, pe)
    return out.reshape(B, S, D)


# speedup vs baseline: 1.6421x; 1.6421x over previous
import functools

import numpy as np

import jax
import jax.numpy as jnp
from jax import lax
from jax.experimental import pallas as pl
from jax.experimental.pallas import tpu as pltpu
from jax.experimental.pallas import tpu_sc as plsc

VOCAB = 100000
D = 768
B = 4
S = 2048
N = B * S
NC = 2
NS = 16
NW = NC * NS
TPW = N // NW
CH = 32
NCH = TPW // CH
LANES = 16
VEC = D // LANES


@functools.lru_cache(maxsize=1)
def _positional_encoding(max_seq, d_model):
    pos = np.arange(max_seq, dtype=np.float32)[:, None]
    i = np.arange(0, d_model, 2, dtype=np.float32)[None, :]
    denom = np.power(np.float32(10000.0), i / np.float32(d_model))
    arg = (pos / denom).astype(np.float32)
    pe = np.stack([np.sin(arg), np.cos(arg)], axis=2).astype(np.float32)
    return jnp.asarray(pe.reshape(max_seq, d_model))


def _body(table, tokens, pe, out, idx_v,
          rows0, rows1, rows2, pe0, pe1,
          sg0, sg1, sg2, sp0, sp1, so0, so1, so2):
    rows = (rows0, rows1, rows2)
    pes = (pe0, pe1)
    sgs = (sg0, sg1, sg2)
    sps = (sp0, sp1)
    sos = (so0, so1, so2)
    wid = lax.axis_index("s") * NC + lax.axis_index("c")
    base = wid * TPW
    pltpu.sync_copy(tokens.at[pl.ds(base, TPW)], idx_v)
    s0 = lax.rem(base, S)

    def start(c):
        ir, ip = c % 3, c % 2
        cb = c * CH
        pltpu.async_copy(table.at[idx_v.at[pl.ds(cb, CH)]], rows[ir], sgs[ir])
        pltpu.async_copy(pe.at[pl.ds(s0 + cb, CH)], pes[ip], sps[ip])

    start(0)
    start(1)
    for c in range(NCH):
        ir, ip = c % 3, c % 2
        cb = c * CH
        pltpu.make_async_copy(table.at[idx_v.at[pl.ds(cb, CH)]],
                              rows[ir], sgs[ir]).wait()
        pltpu.make_async_copy(pe.at[pl.ds(s0 + cb, CH)], pes[ip],
                              sps[ip]).wait()

        rv, pv = rows[ir], pes[ip]

        @plsc.parallel_loop(0, CH, step=1, unroll=2)
        def _add(t):
            for j in range(VEC):
                sl = (t, pl.ds(j * LANES, LANES))
                rv[sl] = rv[sl] + pv[sl]

        pltpu.async_copy(rows[ir], out.at[pl.ds(base + cb, CH)], sos[ir])
        n = c + 2
        if n < NCH:
            jr = n % 3
            if c >= 1:
                pltpu.make_async_copy(
                    rows[jr], out.at[pl.ds(base + (c - 1) * CH, CH)],
                    sos[jr]).wait()
            start(n)
    for k in range(3):
        c = NCH - 3 + k
        pltpu.make_async_copy(rows[c % 3], out.at[pl.ds(base + c * CH, CH)],
                              sos[c % 3]).wait()


@jax.jit
def kernel(tokens, table):
    pe = _positional_encoding(S, D)
    tok = tokens.reshape(N).astype(jnp.int32)
    mesh = plsc.VectorSubcoreMesh(core_axis_name="c", subcore_axis_name="s")
    f = pl.kernel(
        _body,
        out_type=jax.ShapeDtypeStruct((N, D), jnp.float32),
        mesh=mesh,
        scratch_types=[
            pltpu.VMEM((TPW,), jnp.int32),
            pltpu.VMEM((CH, D), jnp.float32),
            pltpu.VMEM((CH, D), jnp.float32),
            pltpu.VMEM((CH, D), jnp.float32),
            pltpu.VMEM((CH, D), jnp.float32),
            pltpu.VMEM((CH, D), jnp.float32),
            pltpu.SemaphoreType.DMA,
            pltpu.SemaphoreType.DMA,
            pltpu.SemaphoreType.DMA,
            pltpu.SemaphoreType.DMA,
            pltpu.SemaphoreType.DMA,
            pltpu.SemaphoreType.DMA,
            pltpu.SemaphoreType.DMA,
            pltpu.SemaphoreType.DMA,
        ],
    )
    out = f(table, tok, pe)
    return out.reshape(B, S, D)
